# baseline (speedup 1.0000x reference)
import functools

import jax
import jax.numpy as jnp
from jax import lax
from jax.experimental import pallas as pl
from jax.experimental.pallas import tpu as pltpu
from jax.experimental.pallas import tpu_sc as plsc

VOCAB = 1000000
EMB_DIM = 64
BATCH = 4096
HIST = 50
N = BATCH * HIST
NW = 32
G = 128
K = 8
NG = N // G
UNITS = NG // K


@functools.partial(
    pl.kernel,
    mesh=plsc.VectorSubcoreMesh(core_axis_name="c", subcore_axis_name="s"),
    out_type=jax.ShapeDtypeStruct((NG, G, EMB_DIM), jnp.float32),
    scratch_types=[
        pltpu.VMEM((K, G), jnp.int32),
        pltpu.VMEM((K, G, EMB_DIM), jnp.float32),
        pltpu.SemaphoreType.DMA,
    ],
    compiler_params=pltpu.CompilerParams(use_tc_tiling_on_sc=False),
)
def _sc_gather(idx_hbm, table_hbm, out_hbm, idx_v, rows_v, sem):
    wid = lax.axis_index("s") * 2 + lax.axis_index("c")
    n_units = lax.div(UNITS - wid + NW - 1, NW)

    def unit_body(i, carry):
        t = wid + i * NW
        grp = t * K
        pltpu.sync_copy(idx_hbm.at[pl.ds(grp, K)], idx_v)
        copies = []
        for j in range(K):
            copies.append(
                pltpu.async_copy(table_hbm.at[idx_v.at[j]], rows_v.at[j], sem)
            )
        for cp in copies:
            cp.wait()
        pltpu.sync_copy(rows_v, out_hbm.at[pl.ds(grp, K)])
        return carry

    lax.fori_loop(0, n_units, unit_body, 0)


def kernel(batch, table):
    idx = batch.reshape(NG, G).astype(jnp.int32)
    out = _sc_gather(idx, table)
    return out.reshape(BATCH, HIST, EMB_DIM)


# speedup vs baseline: 1.0080x; 1.0080x over previous
import functools

import jax
import jax.numpy as jnp
from jax import lax
from jax.experimental import pallas as pl
from jax.experimental.pallas import tpu as pltpu
from jax.experimental.pallas import tpu_sc as plsc

VOCAB = 1000000
EMB_DIM = 64
BATCH = 4096
HIST = 50
N = BATCH * HIST
NW = 32
G = 128
GPW = N // NW // G
K = 5
UPW = GPW // K
UNITS = NW * UPW


@functools.partial(
    pl.kernel,
    mesh=plsc.VectorSubcoreMesh(core_axis_name="c", subcore_axis_name="s"),
    out_type=jax.ShapeDtypeStruct((UNITS, K, G, EMB_DIM), jnp.float32),
    scratch_types=[
        pltpu.VMEM((GPW, G), jnp.int32),
        pltpu.VMEM((K, G, EMB_DIM), jnp.float32),
        pltpu.VMEM((K, G, EMB_DIM), jnp.float32),
        pltpu.SemaphoreType.DMA,
        pltpu.SemaphoreType.DMA,
        pltpu.SemaphoreType.DMA,
        pltpu.SemaphoreType.DMA,
    ],
    compiler_params=pltpu.CompilerParams(use_tc_tiling_on_sc=False),
)
def _sc_gather(idx_hbm, table_hbm, out_hbm, idx_v, rows0, rows1,
               g0, g1, o0, o1):
    wid = lax.axis_index("s") * 2 + lax.axis_index("c")
    rows = (rows0, rows1)
    gsem = (g0, g1)
    osem = (o0, o1)

    pltpu.sync_copy(idx_hbm.at[wid], idx_v)

    gathers = [None, None]
    outs = [None, None]

    def fire_unit(i):
        b = i % 2
        cps = []
        for j in range(K):
            cps.append(
                pltpu.async_copy(
                    table_hbm.at[idx_v.at[i * K + j]], rows[b].at[j], gsem[b]
                )
            )
        gathers[b] = cps

    def retire_unit(i):
        b = i % 2
        for cp in gathers[b]:
            cp.wait()
        gathers[b] = None
        outs[b] = pltpu.async_copy(
            rows[b], out_hbm.at[wid * UPW + i], osem[b]
        )

    for i in range(UPW):
        b = i % 2
        if outs[b] is not None:
            outs[b].wait()
            outs[b] = None
        fire_unit(i)
        if i >= 1:
            retire_unit(i - 1)
    retire_unit(UPW - 1)
    for b in range(2):
        if outs[b] is not None:
            outs[b].wait()


def kernel(batch, table):
    idx = batch.reshape(NW, GPW, G).astype(jnp.int32)
    out = _sc_gather(idx, table)
    return out.reshape(BATCH, HIST, EMB_DIM)
